# baseline (speedup 1.0000x reference)
import dataclasses
import functools

import jax
import jax.numpy as jnp
from jax import lax
from jax.experimental import pallas as pl
from jax.experimental.pallas import tpu as pltpu
from jax.experimental.pallas import tpu_sc as plsc

_N = 10000
_ROWW = 85
_NSUB = 16
_LANES = 16
_CHUNK = 640
_GROUPS = _CHUNK // _LANES
_CHUNKW = _CHUNK * _ROWW


def _sc_reduce(flat):
    mesh = plsc.VectorSubcoreMesh(core_axis_name="c", subcore_axis_name="s")
    cp = pltpu.CompilerParams()
    if "needs_layout_passes" in pltpu.CompilerParams.__dataclass_fields__:
        cp = dataclasses.replace(cp, needs_layout_passes=False)

    @functools.partial(
        pl.kernel,
        out_type=jax.ShapeDtypeStruct((_LANES,), jnp.float32),
        mesh=mesh,
        compiler_params=cp,
        scratch_types=[
            pltpu.VMEM((_CHUNKW,), jnp.float32),
            pltpu.VMEM((_LANES,), jnp.float32),
            pltpu.VMEM((_NSUB * _LANES,), jnp.float32),
            pltpu.VMEM_SHARED((_NSUB * _LANES,), jnp.float32),
        ],
    )
    def k(data_hbm, out_hbm, chunk_v, stage_v, all_v, shared):
        cid = lax.axis_index("c")
        sid = lax.axis_index("s")
        iota = lax.iota(jnp.int32, _LANES)
        i85 = iota * _ROWW

        start_row = sid * _CHUNK
        nrows = jnp.clip(_N - start_row, 0, _CHUNK)

        pltpu.sync_copy(data_hbm.at[pl.ds(start_row * _ROWW, _CHUNKW)],
                        chunk_v)

        zero_f = jnp.zeros((_LANES,), jnp.float32)
        zero_i = jnp.zeros((_LANES,), jnp.int32)

        def grp(g, carry):
            vacc, bb = carry
            idx0 = i85 + g * (_LANES * _ROWW)
            c0 = plsc.load_gather(chunk_v, [idx0])
            c1 = plsc.load_gather(chunk_v, [idx0 + 1])
            c2 = plsc.load_gather(chunk_v, [idx0 + 2])
            c3 = plsc.load_gather(chunk_v, [idx0 + 3])
            c4 = plsc.load_gather(chunk_v, [idx0 + 4])
            s5 = (c0 + c1) + (c2 + c3) + c4
            validb = (g * _LANES + iota) < nrows
            badb = jnp.logical_and(c4 < 0.0, validb)
            bad_i = jnp.where(badb, 1, 0).astype(jnp.int32)
            cs = plsc.cumsum(bad_i)
            keepb = jnp.logical_and(validb, (cs + bb) == 0)
            keep_f = jnp.where(keepb, 1.0, 0.0).astype(jnp.float32)
            cnt = plsc.all_reduce_population_count(badb)
            return (vacc + s5 * keep_f, bb + cnt)

        vacc, bb = lax.fori_loop(0, _GROUPS, grp, (zero_f, zero_i))

        s_w = jnp.sum(vacc)
        a_f = jnp.where(bb == 0, 1.0, 0.0).astype(jnp.float32)
        s_splat = jnp.full((_LANES,), s_w, jnp.float32)
        stage_v[...] = jnp.where(iota == 0, s_splat,
                                 jnp.where(iota == 1, a_f, zero_f))
        pltpu.sync_copy(stage_v, shared.at[pl.ds(sid * _LANES, _LANES)])
        plsc.subcore_barrier()

        @pl.when(jnp.logical_and(cid == 0, sid == 0))
        def _():
            pltpu.sync_copy(shared, all_v)
            svec = plsc.load_gather(all_v, [iota * _LANES])
            avec = plsc.load_gather(all_v, [iota * _LANES + 1])
            badw = jnp.where(avec < 0.5, 1, 0).astype(jnp.int32)
            excl = plsc.cumsum(badw) - badw
            keepw = jnp.where(excl == 0, 1.0, 0.0).astype(jnp.float32)
            tot = jnp.sum(svec * keepw)
            stage_v[...] = jnp.full((_LANES,), tot, jnp.float32)
            pltpu.sync_copy(stage_v, out_hbm)

    return k(flat)


def kernel(data):
    flat = data.reshape(-1)
    out = _sc_reduce(flat)
    return out[0]


# speedup vs baseline: 2.1592x; 2.1592x over previous
import jax
import jax.numpy as jnp
from jax import lax
from jax.experimental import pallas as pl
from jax.experimental.pallas import tpu as pltpu

_N = 10000
_ROWW = 85
_GRID = 10
_BLK = _N // _GRID


def _body(x_ref, o_ref, acc_ref, alive_ref):
    i = pl.program_id(0)

    @pl.when(i == 0)
    def _():
        acc_ref[0, 0] = 0.0
        alive_ref[0, 0] = 1.0

    x = x_ref[0, :, :]
    rowi = lax.broadcasted_iota(jnp.int32, (_BLK, _ROWW), 0)
    lane = lax.broadcasted_iota(jnp.int32, (_BLK, _ROWW), 1)
    conf_bad = jnp.logical_and(lane == 4, x < 0.0)
    r_bad = jnp.min(jnp.where(conf_bad, rowi, _BLK))
    m = jnp.logical_and(lane < 5, rowi < r_bad)
    s_b = jnp.sum(jnp.where(m, x, 0.0))

    alive = alive_ref[0, 0]
    acc = acc_ref[0, 0] + alive * s_b
    acc_ref[0, 0] = acc
    alive_ref[0, 0] = alive * jnp.where(r_bad == _BLK, 1.0, 0.0)

    @pl.when(i == _GRID - 1)
    def _():
        o_ref[...] = jnp.full((1, 1), acc, jnp.float32)


def kernel(data):
    out = pl.pallas_call(
        _body,
        grid=(_GRID,),
        in_specs=[pl.BlockSpec((1, _BLK, _ROWW), lambda i: (0, i, 0))],
        out_specs=pl.BlockSpec((1, 1), lambda i: (0, 0)),
        out_shape=jax.ShapeDtypeStruct((1, 1), jnp.float32),
        scratch_shapes=[pltpu.SMEM((1, 1), jnp.float32),
                        pltpu.SMEM((1, 1), jnp.float32)],
    )(data)
    return out[0, 0]


# speedup vs baseline: 2.3334x; 1.0807x over previous
import jax
import jax.numpy as jnp
from jax import lax
from jax.experimental import pallas as pl
from jax.experimental.pallas import tpu as pltpu

_N = 10000
_ROWW = 85
_GRID = 10
_BLK = _N // _GRID
_FOLD = _BLK // 8


def _body(x_ref, o_ref, vacc_ref, acc_ref, alive_ref):
    i = pl.program_id(0)

    @pl.when(i == 0)
    def _():
        vacc_ref[...] = jnp.zeros((8, _ROWW), jnp.float32)
        acc_ref[0, 0] = 0.0
        alive_ref[0, 0] = 1.0

    lane8 = lax.broadcasted_iota(jnp.int32, (8, _ROWW), 1)

    @pl.when(alive_ref[0, 0] > 0.0)
    def _():
        x = x_ref[0, :, :]
        s = x.reshape(_FOLD, 8, _ROWW)
        addf = jnp.sum(s, axis=0)
        minf = jnp.min(s, axis=0)
        confmin = jnp.min(jnp.where(lane8 == 4, minf, jnp.inf))

        @pl.when(confmin >= 0.0)
        def _():
            vacc_ref[...] = vacc_ref[...] + addf

        @pl.when(confmin < 0.0)
        def _():
            rowi = lax.broadcasted_iota(jnp.int32, (_BLK, _ROWW), 0)
            lane = lax.broadcasted_iota(jnp.int32, (_BLK, _ROWW), 1)
            conf_bad = jnp.logical_and(lane == 4, x < 0.0)
            r_bad = jnp.min(jnp.where(conf_bad, rowi, _BLK))
            m = jnp.logical_and(lane < 5, rowi < r_bad)
            acc_ref[0, 0] = acc_ref[0, 0] + jnp.sum(jnp.where(m, x, 0.0))
            alive_ref[0, 0] = 0.0

    @pl.when(i == _GRID - 1)
    def _():
        tot = acc_ref[0, 0] + jnp.sum(
            jnp.where(lane8 < 5, vacc_ref[...], 0.0))
        o_ref[...] = jnp.full((1, 1), tot, jnp.float32)


def kernel(data):
    out = pl.pallas_call(
        _body,
        grid=(_GRID,),
        in_specs=[pl.BlockSpec((1, _BLK, _ROWW), lambda i: (0, i, 0))],
        out_specs=pl.BlockSpec((1, 1), lambda i: (0, 0)),
        out_shape=jax.ShapeDtypeStruct((1, 1), jnp.float32),
        scratch_shapes=[pltpu.VMEM((8, _ROWW), jnp.float32),
                        pltpu.SMEM((1, 1), jnp.float32),
                        pltpu.SMEM((1, 1), jnp.float32)],
    )(data)
    return out[0, 0]
